# Initial kernel scaffold; baseline (speedup 1.0000x reference)
import jax
import jax.numpy as jnp
from jax.experimental import pallas as pl


def kernel(region_true, affinity_true, region_pred, affinity_pred, confidence, fg_mask, bg_mask):
    raise NotImplementedError("write your pallas kernel here")


# speedup vs baseline: 20.1502x; 20.1502x over previous
import jax
import jax.numpy as jnp
from jax import lax
from jax.experimental import pallas as pl
from jax.experimental.pallas import tpu as pltpu

B, H, W = 16, 384, 384
EPS = 1e-7
HI_BITS = 0x7F800001


def _loss_kernel(rt_ref, at_ref, rp_ref, ap_ref, cf_ref, fg_ref, bg_ref,
                 out_ref, acc_ref):
    i = pl.program_id(0)

    @pl.when(i == 0)
    def _init():
        acc_ref[0] = 0.0
        acc_ref[1] = 0.0

    rt = rt_ref[0]
    at = at_ref[0]
    rp = rp_ref[0]
    ap = ap_ref[0]
    cf = cf_ref[0]
    fg = fg_ref[0]
    bg = bg_ref[0]

    dr = rt - rp
    da = at - ap
    l_total = (dr * dr + da * da) * cf

    fg_num = jnp.sum(fg)
    bg_num = jnp.sum(bg).astype(jnp.int32)
    neg_num = jnp.minimum(
        bg_num, jnp.maximum((fg_num * 3.0).astype(jnp.int32), 10000))

    keys = jnp.where(bg > 0.0,
                     lax.bitcast_convert_type(l_total, jnp.int32),
                     jnp.int32(-1))

    def bisect(_, carry):
        lo, hi = carry
        mid = lo + (hi - lo) // 2
        cnt = jnp.sum(jnp.where(keys >= mid, 1, 0))
        take = cnt >= neg_num
        return jnp.where(take, mid, lo), jnp.where(take, hi, mid)

    kth, _ = lax.fori_loop(0, 31, bisect,
                           (jnp.int32(0), jnp.int32(HI_BITS)))

    hard = (keys >= kth).astype(jnp.float32)
    num = jnp.sum(l_total * (hard + fg))
    den = jnp.sum(cf * (hard + fg))

    acc_ref[0] += num
    acc_ref[1] += den

    @pl.when(i == B - 1)
    def _fin():
        out_ref[...] = jnp.full((1, 1), acc_ref[0] / (acc_ref[1] + EPS),
                                dtype=jnp.float32)


def kernel(region_true, affinity_true, region_pred, affinity_pred,
           confidence, fg_mask, bg_mask):
    spec = pl.BlockSpec((1, H, W), lambda i: (i, 0, 0))
    out = pl.pallas_call(
        _loss_kernel,
        grid=(B,),
        in_specs=[spec] * 7,
        out_specs=pl.BlockSpec((1, 1), lambda i: (0, 0)),
        out_shape=jax.ShapeDtypeStruct((1, 1), jnp.float32),
        scratch_shapes=[pltpu.SMEM((2,), jnp.float32)],
    )(region_true, affinity_true, region_pred, affinity_pred,
      confidence, fg_mask, bg_mask)
    return out[0, 0]


# speedup vs baseline: 47.1697x; 2.3409x over previous
import jax
import jax.numpy as jnp
from jax import lax
from jax.experimental import pallas as pl
from jax.experimental.pallas import tpu as pltpu

B, H, W = 16, 384, 384
EPS = 1e-7
HI_BITS = 0x7F800001


def _loss_kernel(rt_ref, at_ref, rp_ref, ap_ref, cf_ref, fg_ref, bg_ref,
                 out_ref, keys_ref, conf_ref, k_ref, acc_ref):
    i = pl.program_id(0)

    @pl.when(i == 0)
    def _init():
        acc_ref[0] = 0.0
        acc_ref[1] = 0.0

    @pl.when(i < B)
    def _phase1():
        rt = rt_ref[0]
        at = at_ref[0]
        rp = rp_ref[0]
        ap = ap_ref[0]
        cf = cf_ref[0]
        fg = fg_ref[0]
        bg = bg_ref[0]

        dr = rt - rp
        da = at - ap
        l_total = (dr * dr + da * da) * cf

        fg_num = jnp.sum(fg)
        bg_num = jnp.sum(bg).astype(jnp.int32)
        neg_num = jnp.minimum(
            bg_num, jnp.maximum((fg_num * 3.0).astype(jnp.int32), 10000))

        keys = jnp.where(bg > 0.0,
                         lax.bitcast_convert_type(l_total, jnp.int32),
                         jnp.int32(-1))

        keys_ref[pl.ds(i, 1)] = keys[None]
        conf_ref[pl.ds(i, 1)] = cf[None]
        k_ref[pl.ds(i, 1)] = jnp.full((1, 1, 1), neg_num, dtype=jnp.int32)

        acc_ref[0] += jnp.sum(l_total * fg)
        acc_ref[1] += jnp.sum(cf * fg)

    @pl.when(i == B)
    def _phase2():
        keys = keys_ref[...]
        k = k_ref[...]

        def bisect(_, carry):
            lo, hi = carry
            mid = lo + (hi - lo) // 2
            cnt = jnp.sum((keys >= mid).astype(jnp.int32), axis=(1, 2),
                          keepdims=True)
            take = cnt >= k
            return jnp.where(take, mid, lo), jnp.where(take, hi, mid)

        kth, _ = lax.fori_loop(
            0, 31, bisect,
            (jnp.zeros((B, 1, 1), jnp.int32),
             jnp.full((B, 1, 1), HI_BITS, jnp.int32)))

        hard = keys >= kth
        l_vals = lax.bitcast_convert_type(keys, jnp.float32)
        num = jnp.sum(jnp.where(hard, l_vals, 0.0))
        den = jnp.sum(jnp.where(hard, conf_ref[...], 0.0))

        out_ref[...] = jnp.full(
            (1, 1), (acc_ref[0] + num) / (acc_ref[1] + den + EPS),
            dtype=jnp.float32)


def kernel(region_true, affinity_true, region_pred, affinity_pred,
           confidence, fg_mask, bg_mask):
    spec = pl.BlockSpec((1, H, W), lambda i: (jnp.minimum(i, B - 1), 0, 0))
    out = pl.pallas_call(
        _loss_kernel,
        grid=(B + 1,),
        in_specs=[spec] * 7,
        out_specs=pl.BlockSpec((1, 1), lambda i: (0, 0)),
        out_shape=jax.ShapeDtypeStruct((1, 1), jnp.float32),
        scratch_shapes=[
            pltpu.VMEM((B, H, W), jnp.int32),
            pltpu.VMEM((B, H, W), jnp.float32),
            pltpu.VMEM((B, 1, 1), jnp.int32),
            pltpu.SMEM((2,), jnp.float32),
        ],
    )(region_true, affinity_true, region_pred, affinity_pred,
      confidence, fg_mask, bg_mask)
    return out[0, 0]
